# baseline (speedup 1.0000x reference)
import functools

import jax
import jax.numpy as jnp
from jax import lax
from jax.experimental import pallas as pl
from jax.experimental.pallas import tpu as pltpu
from jax.experimental.pallas import tpu_sc as plsc

N = 1_000_000
CR = 2000
CE = CR * 4
NCHUNKS = N // CR
NW = 32
EPSILON = 1e-7

_mesh = plsc.VectorSubcoreMesh(core_axis_name="c", subcore_axis_name="s")


@functools.partial(
    pl.kernel,
    out_type=jax.ShapeDtypeStruct((NW, 48), jnp.float32),
    mesh=_mesh,
    scratch_types=[
        pltpu.VMEM((CE,), jnp.float32),
        pltpu.VMEM((CE,), jnp.float32),
        pltpu.VMEM((CR,), jnp.int32),
        pltpu.VMEM((48,), jnp.float32),
    ],
)
def _loss_partials(t_hbm, r_hbm, lab_hbm, out_hbm, tv, rv, lv, accv):
    wid = lax.axis_index("s") * 2 + lax.axis_index("c")
    iota = lax.iota(jnp.int32, 16)
    row_rep = iota >> 2
    zero = jnp.zeros((16,), jnp.float32)
    one = jnp.ones((16,), jnp.float32)

    nch = jnp.where(wid < (NCHUNKS % NW), NCHUNKS // NW + 1, NCHUNKS // NW)

    def chunk_body(k, carry):
        c = wid + k * NW
        pltpu.sync_copy(t_hbm.at[pl.ds(c * CE, CE)], tv)
        pltpu.sync_copy(r_hbm.at[pl.ds(c * CE, CE)], rv)
        pltpu.sync_copy(lab_hbm.at[pl.ds(c * CR, CR)], lv)

        def gstep(q, acc):
            acc_a, acc_v, acc_p = acc
            lab16 = lv[pl.ds(q * 16, 16)]
            acc_v = acc_v + jnp.where(lab16 != -1, one, zero)
            acc_p = acc_p + jnp.where(lab16 == 1, one, zero)
            base = q * 64
            for m in range(4):
                labx = lab16.at[m * 4 + row_rep].get(mode="promise_in_bounds")
                t = tv[pl.ds(base + m * 16, 16)]
                r = rv[pl.ds(base + m * 16, 16)]
                x = t - r
                ax = jnp.abs(x)
                sl1 = jnp.where(ax <= 1.0, 0.5 * (x * x), ax - 0.5)
                acc_a = acc_a + jnp.where(labx == 1, sl1, zero)
            return acc_a, acc_v, acc_p

        return lax.fori_loop(0, CR // 16, gstep, carry)

    acc_a, acc_v, acc_p = lax.fori_loop(0, nch, chunk_body, (zero, zero, zero))
    accv[pl.ds(0, 16)] = acc_a
    accv[pl.ds(16, 16)] = acc_v
    accv[pl.ds(32, 16)] = acc_p
    pltpu.sync_copy(accv, out_hbm.at[wid])


def kernel(rpn_bbox_targets, rpn_regression, rpn_labels):
    t = jnp.reshape(rpn_bbox_targets, (-1,))
    r = jnp.reshape(rpn_regression, (-1,))
    parts = _loss_partials(t, r, rpn_labels)
    a = jnp.sum(parts[:, 0:16])
    nvalid = jnp.sum(parts[:, 16:32])
    npos = jnp.sum(parts[:, 32:48])
    b = nvalid * EPSILON + npos
    return a / b
